# baseline (speedup 1.0000x reference)
import functools

import jax
import jax.numpy as jnp
from jax import lax
from jax.experimental import pallas as pl
from jax.experimental.pallas import tpu as pltpu
from jax.experimental.pallas import tpu_sc as plsc

NG = 4
R = 128
C = 128
N = R * C

_NW = 32
_CHUNK = N // _NW
_NSTREAM = _CHUNK // 128


def _iotas():
    row = lax.broadcasted_iota(jnp.int32, (R, C), 0)
    col = lax.broadcasted_iota(jnp.int32, (R, C), 1)
    return row, col


def _cumsum2d(x):
    ii = lax.broadcasted_iota(jnp.int32, (R, R), 0)
    jj = lax.broadcasted_iota(jnp.int32, (R, R), 1)
    upper_incl = (ii <= jj).astype(jnp.float32)
    lower_strict = (jj < ii).astype(jnp.float32)
    rowc = jnp.dot(x, upper_incl, preferred_element_type=jnp.float32)
    totals = rowc[:, C - 1:C]
    prefix = jnp.dot(lower_strict, totals, preferred_element_type=jnp.float32)
    return rowc + prefix


def _sort_kernel(pred_ref, grp_ref, vs_ref, idx_ref, payload_ref):
    v = pred_ref[...]
    g = grp_ref[...].astype(jnp.float32)
    row, col = _iotas()

    masks = [(g == float(gi)).astype(jnp.float32) for gi in range(NG)]
    cums = [_cumsum2d(m) for m in masks]
    counts = [jnp.sum(m) for m in masks]
    max_len = counts[0]
    for gi in range(1, NG):
        max_len = jnp.maximum(max_len, counts[gi])

    weight = jnp.zeros((R, C), jnp.float32)
    for gi in range(NG):
        n = jnp.maximum(counts[gi], 1.0)
        q = jnp.floor(max_len / n)
        q = jnp.where(q * n > max_len, q - 1.0, q)
        q = jnp.where((q + 1.0) * n <= max_len, q + 1.0, q)
        r_extra = max_len - q * n
        w_g = q + (cums[gi] - 1.0 < r_extra).astype(jnp.float32)
        weight = weight + masks[gi] * w_g

    payload_ref[...] = g * 65536.0 + weight

    idxf = (row * C + col).astype(jnp.float32)

    def partner_of(x, bit_set, dist, axis):
        size = (R, C)[axis]
        fwd = pltpu.roll(x, size - dist, axis)
        bwd = pltpu.roll(x, dist, axis)
        return jnp.where(bit_set, bwd, fwd)

    for K in range(1, 15):
        if K < 7:
            dirbit = (col >> K) & 1
        elif K < 14:
            dirbit = (row >> (K - 7)) & 1
        else:
            dirbit = jnp.zeros((R, C), jnp.int32)
        up = dirbit == 0
        for j in range(K - 1, -1, -1):
            if j < 7:
                bit = ((col >> j) & 1) == 1
                axis, dist = 1, 1 << j
            else:
                bit = ((row >> (j - 7)) & 1) == 1
                axis, dist = 0, 1 << (j - 7)
            pv = partner_of(v, bit, dist, axis)
            pw = partner_of(idxf, bit, dist, axis)
            keep_min = up ^ bit
            take = (keep_min & (pv < v)) | (~keep_min & (pv > v))
            v = jnp.where(take, pv, v)
            idxf = jnp.where(take, pw, idxf)

    vs_ref[...] = v
    idx_ref[...] = idxf.astype(jnp.int32)


def _finish_kernel(vs_ref, ps_ref, out_ref):
    v = vs_ref[...]
    payload = ps_ref[...]
    row, col = _iotas()

    g_sorted = jnp.floor(payload * (1.0 / 65536.0))
    w_sorted = payload - g_sorted * 65536.0

    max_len = jnp.zeros((), jnp.float32)
    cw = []
    for gi in range(NG):
        m = (g_sorted == float(gi)).astype(jnp.float32)
        max_len = jnp.maximum(max_len, jnp.sum(m))
        cw.append(_cumsum2d(w_sorted * m))

    s_abs = jnp.zeros((R, C), jnp.float32)
    for i in range(NG - 1):
        for j in range(i + 1, NG):
            s_abs = s_abs + jnp.abs(cw[i] - cw[j])

    nxt_lane = pltpu.roll(v, C - 1, 1)
    nxt_row = pltpu.roll(v, R - 1, 0)
    v_next = jnp.where(col == C - 1, nxt_row[:, 0:1], nxt_lane)
    dv = v_next - v
    total = jnp.sum(dv * s_abs)

    npairs = NG * (NG - 1) // 2
    out_ref[...] = jnp.broadcast_to(total / (float(npairs) * max_len), (1, 1))


def _sc_gather_body(payload_hbm, idx_hbm, out_hbm, idx_v, vals_v, sem):
    wid = lax.axis_index("s") * 2 + lax.axis_index("c")
    pltpu.sync_copy(idx_hbm.at[wid], idx_v)
    for j in range(_NSTREAM):
        pltpu.async_copy(payload_hbm.at[idx_v.at[j]], vals_v.at[j], sem).wait()
    pltpu.sync_copy(vals_v, out_hbm.at[wid])


def _gather_sorted(payload_flat, idx3):
    mesh = plsc.VectorSubcoreMesh(core_axis_name="c", subcore_axis_name="s")
    run = functools.partial(
        pl.kernel,
        mesh=mesh,
        out_type=jax.ShapeDtypeStruct((_NW, _NSTREAM, 128), jnp.float32),
        scratch_types=[
            pltpu.VMEM((_NSTREAM, 128), jnp.int32),
            pltpu.VMEM((_NSTREAM, 128), jnp.float32),
            pltpu.SemaphoreType.DMA,
        ],
    )(_sc_gather_body)
    return run(payload_flat, idx3)


def kernel(batch_pred, batch_group):
    v2 = batch_pred.reshape(R, C)
    g2 = batch_group.reshape(R, C)

    vs, idx, payload = pl.pallas_call(
        _sort_kernel,
        out_shape=[
            jax.ShapeDtypeStruct((R, C), jnp.float32),
            jax.ShapeDtypeStruct((R, C), jnp.int32),
            jax.ShapeDtypeStruct((R, C), jnp.float32),
        ],
        in_specs=[
            pl.BlockSpec(memory_space=pltpu.VMEM),
            pl.BlockSpec(memory_space=pltpu.VMEM),
        ],
        out_specs=[pl.BlockSpec(memory_space=pltpu.VMEM)] * 3,
    )(v2, g2)

    payload_sorted = _gather_sorted(
        payload.reshape(N), idx.reshape(_NW, _NSTREAM, 128))

    out = pl.pallas_call(
        _finish_kernel,
        out_shape=jax.ShapeDtypeStruct((1, 1), jnp.float32),
        in_specs=[
            pl.BlockSpec(memory_space=pltpu.VMEM),
            pl.BlockSpec(memory_space=pltpu.VMEM),
        ],
        out_specs=pl.BlockSpec(memory_space=pltpu.VMEM),
    )(vs, payload_sorted.reshape(R, C))
    return out[0, 0]


# speedup vs baseline: 1.0438x; 1.0438x over previous
import functools

import jax
import jax.numpy as jnp
from jax import lax
from jax.experimental import pallas as pl
from jax.experimental.pallas import tpu as pltpu
from jax.experimental.pallas import tpu_sc as plsc

NG = 4
R = 128
C = 128
N = R * C

_NW = 32
_CHUNK = N // _NW
_NSTREAM = _CHUNK // 128


def _iotas():
    row = lax.broadcasted_iota(jnp.int32, (R, C), 0)
    col = lax.broadcasted_iota(jnp.int32, (R, C), 1)
    return row, col


def _cumsum2d(x):
    ii = lax.broadcasted_iota(jnp.int32, (R, R), 0)
    jj = lax.broadcasted_iota(jnp.int32, (R, R), 1)
    upper_incl = (ii <= jj).astype(jnp.float32)
    lower_strict = (jj < ii).astype(jnp.float32)
    rowc = jnp.dot(x, upper_incl, preferred_element_type=jnp.float32)
    totals = rowc[:, C - 1:C]
    prefix = jnp.dot(lower_strict, totals, preferred_element_type=jnp.float32)
    return rowc + prefix


def _sort_kernel(pred_ref, grp_ref, vs_ref, idx_ref, payload_ref):
    v = pred_ref[...]
    g = grp_ref[...].astype(jnp.float32)
    row, col = _iotas()

    masks = [(g == float(gi)).astype(jnp.float32) for gi in range(NG)]
    cums = [_cumsum2d(m) for m in masks]
    counts = [jnp.sum(m) for m in masks]
    max_len = counts[0]
    for gi in range(1, NG):
        max_len = jnp.maximum(max_len, counts[gi])

    weight = jnp.zeros((R, C), jnp.float32)
    for gi in range(NG):
        n = jnp.maximum(counts[gi], 1.0)
        q = jnp.floor(max_len / n)
        q = jnp.where(q * n > max_len, q - 1.0, q)
        q = jnp.where((q + 1.0) * n <= max_len, q + 1.0, q)
        r_extra = max_len - q * n
        w_g = q + (cums[gi] - 1.0 < r_extra).astype(jnp.float32)
        weight = weight + masks[gi] * w_g

    payload_ref[...] = g * 65536.0 + weight

    idxf = (row * C + col).astype(jnp.float32)

    def partner_of(x, bit_set, dist, axis):
        size = (R, C)[axis]
        fwd = pltpu.roll(x, size - dist, axis)
        bwd = pltpu.roll(x, dist, axis)
        return jnp.where(bit_set, bwd, fwd)

    for K in range(1, 15):
        if K < 7:
            dirbit = (col >> K) & 1
        elif K < 14:
            dirbit = (row >> (K - 7)) & 1
        else:
            dirbit = jnp.zeros((R, C), jnp.int32)
        up = dirbit == 0
        for j in range(K - 1, -1, -1):
            if j < 7:
                bit = ((col >> j) & 1) == 1
                axis, dist = 1, 1 << j
            else:
                bit = ((row >> (j - 7)) & 1) == 1
                axis, dist = 0, 1 << (j - 7)
            pv = partner_of(v, bit, dist, axis)
            pw = partner_of(idxf, bit, dist, axis)
            keep_min = up ^ bit
            take = (keep_min & (pv < v)) | (~keep_min & (pv > v))
            v = jnp.where(take, pv, v)
            idxf = jnp.where(take, pw, idxf)

    vs_ref[...] = v
    idx_ref[...] = idxf.astype(jnp.int32)


def _finish_kernel(vs_ref, ps_ref, out_ref):
    v = vs_ref[...]
    payload = ps_ref[...]
    row, col = _iotas()

    g_sorted = jnp.floor(payload * (1.0 / 65536.0))
    w_sorted = payload - g_sorted * 65536.0

    max_len = jnp.zeros((), jnp.float32)
    cw = []
    for gi in range(NG):
        m = (g_sorted == float(gi)).astype(jnp.float32)
        max_len = jnp.maximum(max_len, jnp.sum(m))
        cw.append(_cumsum2d(w_sorted * m))

    s_abs = jnp.zeros((R, C), jnp.float32)
    for i in range(NG - 1):
        for j in range(i + 1, NG):
            s_abs = s_abs + jnp.abs(cw[i] - cw[j])

    nxt_lane = pltpu.roll(v, C - 1, 1)
    nxt_row = pltpu.roll(v, R - 1, 0)
    v_next = jnp.where(col == C - 1, nxt_row[:, 0:1], nxt_lane)
    dv = v_next - v
    total = jnp.sum(dv * s_abs)

    npairs = NG * (NG - 1) // 2
    out_ref[...] = jnp.broadcast_to(total / (float(npairs) * max_len), (1, 1))


def _sc_gather_body(payload_hbm, idx_hbm, out_hbm, idx_v, vals_v, sem):
    wid = lax.axis_index("s") * 2 + lax.axis_index("c")
    pltpu.sync_copy(idx_hbm.at[wid], idx_v)
    copies = [
        pltpu.async_copy(payload_hbm.at[idx_v.at[j]], vals_v.at[j], sem)
        for j in range(_NSTREAM)
    ]
    for c in copies:
        c.wait()
    pltpu.sync_copy(vals_v, out_hbm.at[wid])


def _gather_sorted(payload_flat, idx3):
    mesh = plsc.VectorSubcoreMesh(core_axis_name="c", subcore_axis_name="s")
    run = functools.partial(
        pl.kernel,
        mesh=mesh,
        out_type=jax.ShapeDtypeStruct((_NW, _NSTREAM, 128), jnp.float32),
        scratch_types=[
            pltpu.VMEM((_NSTREAM, 128), jnp.int32),
            pltpu.VMEM((_NSTREAM, 128), jnp.float32),
            pltpu.SemaphoreType.DMA,
        ],
    )(_sc_gather_body)
    return run(payload_flat, idx3)


def kernel(batch_pred, batch_group):
    v2 = batch_pred.reshape(R, C)
    g2 = batch_group.reshape(R, C)

    vs, idx, payload = pl.pallas_call(
        _sort_kernel,
        out_shape=[
            jax.ShapeDtypeStruct((R, C), jnp.float32),
            jax.ShapeDtypeStruct((R, C), jnp.int32),
            jax.ShapeDtypeStruct((R, C), jnp.float32),
        ],
        in_specs=[
            pl.BlockSpec(memory_space=pltpu.VMEM),
            pl.BlockSpec(memory_space=pltpu.VMEM),
        ],
        out_specs=[pl.BlockSpec(memory_space=pltpu.VMEM)] * 3,
    )(v2, g2)

    payload_sorted = _gather_sorted(
        payload.reshape(N), idx.reshape(_NW, _NSTREAM, 128))

    out = pl.pallas_call(
        _finish_kernel,
        out_shape=jax.ShapeDtypeStruct((1, 1), jnp.float32),
        in_specs=[
            pl.BlockSpec(memory_space=pltpu.VMEM),
            pl.BlockSpec(memory_space=pltpu.VMEM),
        ],
        out_specs=pl.BlockSpec(memory_space=pltpu.VMEM),
    )(vs, payload_sorted.reshape(R, C))
    return out[0, 0]


# speedup vs baseline: 1.0601x; 1.0156x over previous
import functools

import jax
import jax.numpy as jnp
from jax import lax
from jax.experimental import pallas as pl
from jax.experimental.pallas import tpu as pltpu
from jax.experimental.pallas import tpu_sc as plsc

NG = 4
R = 128
C = 128
N = R * C

_NW = 32
_CHUNK = N // _NW
_NSTREAM = _CHUNK // 128


def _iotas():
    row = lax.broadcasted_iota(jnp.int32, (R, C), 0)
    col = lax.broadcasted_iota(jnp.int32, (R, C), 1)
    return row, col


def _cumsum2d(x):
    ii = lax.broadcasted_iota(jnp.int32, (R, R), 0)
    jj = lax.broadcasted_iota(jnp.int32, (R, R), 1)
    upper_incl = (ii <= jj).astype(jnp.float32)
    lower_strict = (jj < ii).astype(jnp.float32)
    rowc = jnp.dot(x, upper_incl, preferred_element_type=jnp.float32)
    totals = rowc[:, C - 1:C]
    prefix = jnp.dot(lower_strict, totals, preferred_element_type=jnp.float32)
    return rowc + prefix


def _sort_kernel(pred_ref, grp_ref, vs_ref, idx_ref, payload_ref):
    v = pred_ref[...]
    g = grp_ref[...].astype(jnp.float32)
    row, col = _iotas()

    masks = [(g == float(gi)).astype(jnp.float32) for gi in range(NG)]
    cums = [_cumsum2d(m) for m in masks]
    counts = [jnp.sum(m) for m in masks]
    max_len = counts[0]
    for gi in range(1, NG):
        max_len = jnp.maximum(max_len, counts[gi])

    weight = jnp.zeros((R, C), jnp.float32)
    for gi in range(NG):
        n = jnp.maximum(counts[gi], 1.0)
        q = jnp.floor(max_len / n)
        q = jnp.where(q * n > max_len, q - 1.0, q)
        q = jnp.where((q + 1.0) * n <= max_len, q + 1.0, q)
        r_extra = max_len - q * n
        w_g = q + (cums[gi] - 1.0 < r_extra).astype(jnp.float32)
        weight = weight + masks[gi] * w_g

    payload_ref[...] = g * 65536.0 + weight

    idxf = (row * C + col).astype(jnp.float32)

    def partner_of(x, bit_set, dist, axis):
        size = (R, C)[axis]
        fwd = pltpu.roll(x, size - dist, axis)
        bwd = pltpu.roll(x, dist, axis)
        return jnp.where(bit_set, bwd, fwd)

    for K in range(1, 15):
        if K < 7:
            dirbit = (col >> K) & 1
        elif K < 14:
            dirbit = (row >> (K - 7)) & 1
        else:
            dirbit = jnp.zeros((R, C), jnp.int32)
        up = dirbit == 0
        for j in range(K - 1, -1, -1):
            if j < 7:
                bit = ((col >> j) & 1) == 1
                axis, dist = 1, 1 << j
            else:
                bit = ((row >> (j - 7)) & 1) == 1
                axis, dist = 0, 1 << (j - 7)
            pv = partner_of(v, bit, dist, axis)
            pw = partner_of(idxf, bit, dist, axis)
            keep_min = up ^ bit
            take = (keep_min & (pv < v)) | (~keep_min & (pv > v))
            v = jnp.where(take, pv, v)
            idxf = jnp.where(take, pw, idxf)

    vs_ref[...] = v
    idx_ref[...] = idxf.astype(jnp.int32)


def _finish_kernel(vs_ref, ps_ref, out_ref):
    v = vs_ref[...]
    payload = ps_ref[...]
    row, col = _iotas()

    g_sorted = jnp.floor(payload * (1.0 / 65536.0))
    w_sorted = payload - g_sorted * 65536.0

    max_len = jnp.zeros((), jnp.float32)
    cw = []
    for gi in range(NG):
        m = (g_sorted == float(gi)).astype(jnp.float32)
        max_len = jnp.maximum(max_len, jnp.sum(m))
        cw.append(_cumsum2d(w_sorted * m))

    s_abs = jnp.zeros((R, C), jnp.float32)
    for i in range(NG - 1):
        for j in range(i + 1, NG):
            s_abs = s_abs + jnp.abs(cw[i] - cw[j])

    nxt_lane = pltpu.roll(v, C - 1, 1)
    nxt_row = pltpu.roll(v, R - 1, 0)
    v_next = jnp.where(col == C - 1, nxt_row[:, 0:1], nxt_lane)
    dv = v_next - v
    total = jnp.sum(dv * s_abs)

    npairs = NG * (NG - 1) // 2
    out_ref[...] = jnp.broadcast_to(total / (float(npairs) * max_len), (1, 1))


def _sc_gather_body(payload_hbm, idx_hbm, out_hbm, idx_v, vals_v, stage, sem):
    sid = lax.axis_index("s")
    wid = sid * 2 + lax.axis_index("c")

    @pl.when(sid == 0)
    def _():
        pltpu.sync_copy(payload_hbm, stage)

    pltpu.sync_copy(idx_hbm.at[wid], idx_v)
    plsc.subcore_barrier()
    copies = [
        pltpu.async_copy(stage.at[idx_v.at[j]], vals_v.at[j], sem)
        for j in range(_NSTREAM)
    ]
    for c in copies:
        c.wait()
    pltpu.sync_copy(vals_v, out_hbm.at[wid])


def _gather_sorted(payload_flat, idx3):
    mesh = plsc.VectorSubcoreMesh(core_axis_name="c", subcore_axis_name="s")
    run = functools.partial(
        pl.kernel,
        mesh=mesh,
        out_type=jax.ShapeDtypeStruct((_NW, _NSTREAM, 128), jnp.float32),
        scratch_types=[
            pltpu.VMEM((_NSTREAM, 128), jnp.int32),
            pltpu.VMEM((_NSTREAM, 128), jnp.float32),
            pltpu.MemorySpace.VMEM_SHARED((N,), jnp.float32),
            pltpu.SemaphoreType.DMA,
        ],
    )(_sc_gather_body)
    return run(payload_flat, idx3)


def kernel(batch_pred, batch_group):
    v2 = batch_pred.reshape(R, C)
    g2 = batch_group.reshape(R, C)

    vs, idx, payload = pl.pallas_call(
        _sort_kernel,
        out_shape=[
            jax.ShapeDtypeStruct((R, C), jnp.float32),
            jax.ShapeDtypeStruct((R, C), jnp.int32),
            jax.ShapeDtypeStruct((R, C), jnp.float32),
        ],
        in_specs=[
            pl.BlockSpec(memory_space=pltpu.VMEM),
            pl.BlockSpec(memory_space=pltpu.VMEM),
        ],
        out_specs=[pl.BlockSpec(memory_space=pltpu.VMEM)] * 3,
    )(v2, g2)

    payload_sorted = _gather_sorted(
        payload.reshape(N), idx.reshape(_NW, _NSTREAM, 128))

    out = pl.pallas_call(
        _finish_kernel,
        out_shape=jax.ShapeDtypeStruct((1, 1), jnp.float32),
        in_specs=[
            pl.BlockSpec(memory_space=pltpu.VMEM),
            pl.BlockSpec(memory_space=pltpu.VMEM),
        ],
        out_specs=pl.BlockSpec(memory_space=pltpu.VMEM),
    )(vs, payload_sorted.reshape(R, C))
    return out[0, 0]


# speedup vs baseline: 1.1321x; 1.0680x over previous
import functools

import jax
import jax.numpy as jnp
from jax import lax
from jax.experimental import pallas as pl
from jax.experimental.pallas import tpu as pltpu
from jax.experimental.pallas import tpu_sc as plsc

NG = 4
R = 128
C = 128
N = R * C

_NW = 32
_CHUNK = N // _NW
_NSTREAM = _CHUNK // 128


def _iotas():
    row = lax.broadcasted_iota(jnp.int32, (R, C), 0)
    col = lax.broadcasted_iota(jnp.int32, (R, C), 1)
    return row, col


def _cumsum2d(x):
    ii = lax.broadcasted_iota(jnp.int32, (R, R), 0)
    jj = lax.broadcasted_iota(jnp.int32, (R, R), 1)
    upper_incl = (ii <= jj).astype(jnp.float32)
    lower_strict = (jj < ii).astype(jnp.float32)
    rowc = jnp.dot(x, upper_incl, preferred_element_type=jnp.float32)
    totals = rowc[:, C - 1:C]
    prefix = jnp.dot(lower_strict, totals, preferred_element_type=jnp.float32)
    return rowc + prefix


def _sort_kernel(pred_ref, grp_ref, vs_ref, idx_ref, payload_ref):
    v = pred_ref[...]
    g = grp_ref[...].astype(jnp.float32)
    row, col = _iotas()

    masks = [(g == float(gi)).astype(jnp.float32) for gi in range(NG)]
    cums = [_cumsum2d(m) for m in masks]
    counts = [jnp.sum(m) for m in masks]
    max_len = counts[0]
    for gi in range(1, NG):
        max_len = jnp.maximum(max_len, counts[gi])

    weight = jnp.zeros((R, C), jnp.float32)
    for gi in range(NG):
        n = jnp.maximum(counts[gi], 1.0)
        q = jnp.floor(max_len / n)
        q = jnp.where(q * n > max_len, q - 1.0, q)
        q = jnp.where((q + 1.0) * n <= max_len, q + 1.0, q)
        r_extra = max_len - q * n
        w_g = q + (cums[gi] - 1.0 < r_extra).astype(jnp.float32)
        weight = weight + masks[gi] * w_g

    payload_ref[...] = g * 65536.0 + weight

    idxf = (row * C + col).astype(jnp.float32)

    def partner_of(x, bit_set, dist, axis):
        size = (R, C)[axis]
        fwd = pltpu.roll(x, size - dist, axis)
        bwd = pltpu.roll(x, dist, axis)
        return jnp.where(bit_set, bwd, fwd)

    for K in range(1, 15):
        if K < 7:
            dir_int = (col >> K) & 1
        elif K < 14:
            dir_int = (row >> (K - 7)) & 1
        else:
            dir_int = jnp.zeros((R, C), jnp.int32)
        for j in range(K - 1, -1, -1):
            if j < 7:
                bit_int = (col >> j) & 1
                axis, dist = 1, 1 << j
            else:
                bit_int = (row >> (j - 7)) & 1
                axis, dist = 0, 1 << (j - 7)
            bit = bit_int == 1
            keep_min = (dir_int ^ bit_int) == 0
            pv = partner_of(v, bit, dist, axis)
            pw = partner_of(idxf, bit, dist, axis)
            cc = jnp.where(keep_min, v, pv)
            dd = jnp.where(keep_min, pv, v)
            take = dd < cc
            v = jnp.where(take, pv, v)
            idxf = jnp.where(take, pw, idxf)

    vs_ref[...] = v
    idx_ref[...] = idxf.astype(jnp.int32)


def _finish_kernel(vs_ref, ps_ref, out_ref):
    v = vs_ref[...]
    payload = ps_ref[...]
    row, col = _iotas()

    g_sorted = jnp.floor(payload * (1.0 / 65536.0))
    w_sorted = payload - g_sorted * 65536.0

    max_len = jnp.zeros((), jnp.float32)
    cw = []
    for gi in range(NG):
        m = (g_sorted == float(gi)).astype(jnp.float32)
        max_len = jnp.maximum(max_len, jnp.sum(m))
        cw.append(_cumsum2d(w_sorted * m))

    s_abs = jnp.zeros((R, C), jnp.float32)
    for i in range(NG - 1):
        for j in range(i + 1, NG):
            s_abs = s_abs + jnp.abs(cw[i] - cw[j])

    nxt_lane = pltpu.roll(v, C - 1, 1)
    nxt_row = pltpu.roll(v, R - 1, 0)
    v_next = jnp.where(col == C - 1, nxt_row[:, 0:1], nxt_lane)
    dv = v_next - v
    total = jnp.sum(dv * s_abs)

    npairs = NG * (NG - 1) // 2
    out_ref[...] = jnp.broadcast_to(total / (float(npairs) * max_len), (1, 1))


def _sc_gather_body(payload_hbm, idx_hbm, out_hbm, idx_v, vals_v, stage, sem):
    sid = lax.axis_index("s")
    wid = sid * 2 + lax.axis_index("c")

    @pl.when(sid == 0)
    def _():
        pltpu.sync_copy(payload_hbm, stage)

    pltpu.sync_copy(idx_hbm.at[wid], idx_v)
    plsc.subcore_barrier()
    copies = [
        pltpu.async_copy(stage.at[idx_v.at[j]], vals_v.at[j], sem)
        for j in range(_NSTREAM)
    ]
    for c in copies:
        c.wait()
    pltpu.sync_copy(vals_v, out_hbm.at[wid])


def _gather_sorted(payload_flat, idx3):
    mesh = plsc.VectorSubcoreMesh(core_axis_name="c", subcore_axis_name="s")
    run = functools.partial(
        pl.kernel,
        mesh=mesh,
        out_type=jax.ShapeDtypeStruct((_NW, _NSTREAM, 128), jnp.float32),
        scratch_types=[
            pltpu.VMEM((_NSTREAM, 128), jnp.int32),
            pltpu.VMEM((_NSTREAM, 128), jnp.float32),
            pltpu.MemorySpace.VMEM_SHARED((N,), jnp.float32),
            pltpu.SemaphoreType.DMA,
        ],
    )(_sc_gather_body)
    return run(payload_flat, idx3)


def kernel(batch_pred, batch_group):
    v2 = batch_pred.reshape(R, C)
    g2 = batch_group.reshape(R, C)

    vs, idx, payload = pl.pallas_call(
        _sort_kernel,
        out_shape=[
            jax.ShapeDtypeStruct((R, C), jnp.float32),
            jax.ShapeDtypeStruct((R, C), jnp.int32),
            jax.ShapeDtypeStruct((R, C), jnp.float32),
        ],
        in_specs=[
            pl.BlockSpec(memory_space=pltpu.VMEM),
            pl.BlockSpec(memory_space=pltpu.VMEM),
        ],
        out_specs=[pl.BlockSpec(memory_space=pltpu.VMEM)] * 3,
    )(v2, g2)

    payload_sorted = _gather_sorted(
        payload.reshape(N), idx.reshape(_NW, _NSTREAM, 128))

    out = pl.pallas_call(
        _finish_kernel,
        out_shape=jax.ShapeDtypeStruct((1, 1), jnp.float32),
        in_specs=[
            pl.BlockSpec(memory_space=pltpu.VMEM),
            pl.BlockSpec(memory_space=pltpu.VMEM),
        ],
        out_specs=pl.BlockSpec(memory_space=pltpu.VMEM),
    )(vs, payload_sorted.reshape(R, C))
    return out[0, 0]
